# baseline (speedup 1.0000x reference)
import functools

import jax
import jax.numpy as jnp
from jax import lax
from jax.experimental import pallas as pl
from jax.experimental.pallas import tpu as pltpu
from jax.experimental.pallas import tpu_sc as plsc

_LANES = 16
_NBUF = 3


def _make_sc_kernel(B, S, D, NC, NS):
    NW = NC * NS
    seq_per_tile = S // NW
    C = min(16, seq_per_tile)
    n_chunks = seq_per_tile // C
    vregs_per_row = D // _LANES

    mesh = plsc.VectorSubcoreMesh(core_axis_name="c", subcore_axis_name="s")

    @functools.partial(
        pl.kernel,
        mesh=mesh,
        out_type=jax.ShapeDtypeStruct((B * S, D), jnp.float32),
        scratch_types=(
            [pltpu.VMEM((C,), jnp.int32) for _ in range(2)]
            + [pltpu.VMEM((C, D), jnp.float32) for _ in range(2)]
            + [pltpu.VMEM((C, D), jnp.float32) for _ in range(_NBUF)]
            + [pltpu.SemaphoreType.DMA for _ in range(2 + 2 * _NBUF)]
        ),
    )
    def sc_kernel(x_hbm, ids_hbm, pos_hbm, out_hbm, *scratch):
        idx_v = scratch[0:2]
        pos_v = scratch[2:4]
        xb = scratch[4:4 + _NBUF]
        sem_p = scratch[4 + _NBUF:6 + _NBUF]
        sem_in = scratch[6 + _NBUF:6 + 2 * _NBUF]
        sem_out = scratch[6 + 2 * _NBUF:6 + 3 * _NBUF]

        wid = lax.axis_index("s") * NC + lax.axis_index("c")
        s0 = wid * seq_per_tile
        steps = [(i, b) for i in range(n_chunks) for b in range(B)]

        def row_base(i, b):
            return b * S + s0 + i * C

        def start_pos_gather(i):
            p = i % 2
            pltpu.sync_copy(ids_hbm.at[pl.ds(s0 + i * C, C)], idx_v[p])
            return pltpu.async_copy(pos_hbm.at[idx_v[p]], pos_v[p], sem_p[p])

        def start_in(k):
            i, b = steps[k]
            buf = k % _NBUF
            return pltpu.async_copy(
                x_hbm.at[pl.ds(row_base(i, b), C)], xb[buf], sem_in[buf])

        in_h = [None] * _NBUF
        out_h = [None] * _NBUF
        pos_h = [None, None]

        pos_h[0] = start_pos_gather(0)
        for k in range(min(_NBUF - 1, len(steps))):
            in_h[k % _NBUF] = start_in(k)

        for k, (i, b) in enumerate(steps):
            cur = k % _NBUF
            if b == 0:
                pos_h[i % 2].wait()
                if i + 1 < n_chunks:
                    pos_h[(i + 1) % 2] = start_pos_gather(i + 1)
            if k + _NBUF - 1 < len(steps):
                nxt = (k + _NBUF - 1) % _NBUF
                if out_h[nxt] is not None:
                    out_h[nxt].wait()
                in_h[nxt] = start_in(k + _NBUF - 1)
            in_h[cur].wait()

            pbuf = pos_v[i % 2]
            xbuf = xb[cur]

            def row_body(r, _, pbuf=pbuf, xbuf=xbuf):
                for j in range(vregs_per_row):
                    v = pbuf[r, pl.ds(j * _LANES, _LANES)]
                    plsc.addupdate(xbuf.at[r, pl.ds(j * _LANES, _LANES)], v)
                return 0

            lax.fori_loop(0, C, row_body, 0)

            out_h[cur] = pltpu.async_copy(
                xbuf, out_hbm.at[pl.ds(row_base(i, b), C)], sem_out[cur])

        for h in out_h:
            if h is not None:
                h.wait()

    return sc_kernel


def kernel(x, pos_table):
    B, S, D = x.shape
    info = plsc.get_sparse_core_info()
    sc = _make_sc_kernel(B, S, D, info.num_cores, info.num_subcores)
    positions = jnp.arange(S, dtype=jnp.int32)
    out = sc(x.reshape(B * S, D), positions, pos_table)
    return out.reshape(B, S, D)


# speedup vs baseline: 1.1961x; 1.1961x over previous
import functools

import jax
import jax.numpy as jnp
from jax import lax
from jax.experimental import pallas as pl
from jax.experimental.pallas import tpu as pltpu
from jax.experimental.pallas import tpu_sc as plsc

_LANES = 16
_NBUF = 3


def _make_sc_kernel(B, S, D, NC, NS):
    NW = NC * NS
    seq_per_tile = S // NW
    C = min(16, seq_per_tile)
    n_chunks = seq_per_tile // C
    vregs_per_row = D // _LANES

    mesh = plsc.VectorSubcoreMesh(core_axis_name="c", subcore_axis_name="s")

    @functools.partial(
        pl.kernel,
        mesh=mesh,
        out_type=jax.ShapeDtypeStruct((B * S, D), jnp.float32),
        scratch_types=(
            [pltpu.VMEM((C,), jnp.int32) for _ in range(2)]
            + [pltpu.VMEM((C, D), jnp.float32) for _ in range(2)]
            + [pltpu.VMEM((C, D), jnp.float32) for _ in range(_NBUF)]
            + [pltpu.SemaphoreType.DMA for _ in range(2 + 2 * _NBUF)]
        ),
    )
    def sc_kernel(x_hbm, ids_hbm, pos_hbm, out_hbm, *scratch):
        idx_v = scratch[0:2]
        pos_v = scratch[2:4]
        xb = scratch[4:4 + _NBUF]
        sem_p = scratch[4 + _NBUF:6 + _NBUF]
        sem_in = scratch[6 + _NBUF:6 + 2 * _NBUF]
        sem_out = scratch[6 + 2 * _NBUF:6 + 3 * _NBUF]

        wid = lax.axis_index("s") * NC + lax.axis_index("c")
        s0 = wid * seq_per_tile
        steps = [(i, b) for i in range(n_chunks) for b in range(B)]

        def row_base(i, b):
            return b * S + s0 + i * C

        def start_pos_gather(i):
            p = i % 2
            pltpu.sync_copy(ids_hbm.at[pl.ds(s0 + i * C, C)], idx_v[p])
            return pltpu.async_copy(pos_hbm.at[idx_v[p]], pos_v[p], sem_p[p])

        def start_in(k):
            i, b = steps[k]
            buf = k % _NBUF
            return pltpu.async_copy(
                x_hbm.at[pl.ds(row_base(i, b), C)], xb[buf], sem_in[buf])

        in_h = [None] * _NBUF
        out_h = [None] * _NBUF
        pos_h = [None, None]

        pos_h[0] = start_pos_gather(0)
        for k in range(min(_NBUF - 1, len(steps))):
            in_h[k % _NBUF] = start_in(k)

        for k, (i, b) in enumerate(steps):
            cur = k % _NBUF
            if b == 0:
                pos_h[i % 2].wait()
                if i + 1 < n_chunks:
                    pos_h[(i + 1) % 2] = start_pos_gather(i + 1)
            if k + _NBUF - 1 < len(steps):
                nxt = (k + _NBUF - 1) % _NBUF
                if out_h[nxt] is not None:
                    out_h[nxt].wait()
                in_h[nxt] = start_in(k + _NBUF - 1)
            in_h[cur].wait()

            pbuf = pos_v[i % 2]
            xbuf = xb[cur]

            @plsc.parallel_loop(0, C * vregs_per_row, unroll=8)
            def vreg_body(t, pbuf=pbuf, xbuf=xbuf):
                r = t // vregs_per_row
                j = t % vregs_per_row
                v = pbuf[r, pl.ds(j * _LANES, _LANES)]
                plsc.addupdate(xbuf.at[r, pl.ds(j * _LANES, _LANES)], v)

            out_h[cur] = pltpu.async_copy(
                xbuf, out_hbm.at[pl.ds(row_base(i, b), C)], sem_out[cur])

        for h in out_h:
            if h is not None:
                h.wait()

    return sc_kernel


def kernel(x, pos_table):
    B, S, D = x.shape
    info = plsc.get_sparse_core_info()
    sc = _make_sc_kernel(B, S, D, info.num_cores, info.num_subcores)
    positions = jnp.arange(S, dtype=jnp.int32)
    out = sc(x.reshape(B * S, D), positions, pos_table)
    return out.reshape(B, S, D)
